# baseline (speedup 1.0000x reference)
import functools

import jax
import jax.numpy as jnp
from jax import lax
from jax.experimental import pallas as pl
from jax.experimental.pallas import tpu as pltpu
from jax.experimental.pallas import tpu_sc as plsc

N = 10000
E = 320000
D = 128
R = 8
NB = 4

NPAD = 10240
NCHUNK = 16
CN = NPAD // NCHUNK
TRASH = R * CN
ACC_ROWS = R * CN + 128
EPT = E // 16
BLK = 2000
NBLK = EPT // BLK
CAP = 2560
WT_ROWS = R * CN // 16
ZROWS = 40
SMASK = (1 << 14) - 1


def _make_sc_body():
    def body(nf_hbm, ep_hbm, a_hbm,
             st0, st1, selr,
             sidx0, aidx0, sidx1, aidx1, sidx2, aidx2, sidx3, aidx3,
             row0, row1, row2, row3, zbuf, acc,
             semt0, semt1, semg0, semg1, semg2, semg3,
             sems0, sems1, sems2, sems3, semz):
        core = lax.axis_index("c")
        tid = lax.axis_index("s")
        ebase = tid * EPT
        slots = [
            (sidx0, aidx0, row0, semg0, sems0),
            (sidx1, aidx1, row1, semg1, sems1),
            (sidx2, aidx2, row2, semg2, sems2),
            (sidx3, aidx3, row3, semg3, sems3),
        ]

        z16f = jnp.zeros((16,), jnp.float32)
        tpad = jnp.full((16,), TRASH << 14, jnp.int32)

        def _zrow(i, c):
            for k in range(8):
                zbuf[i, pl.ds(k * 16, 16)] = z16f
            return c
        lax.fori_loop(0, ZROWS, _zrow, 0)

        def _zero_stripe():
            base = tid * WT_ROWS
            ds = [
                pltpu.async_copy(
                    zbuf, acc.at[pl.ds(base + k * ZROWS, ZROWS)], semz)
                for k in range(WT_ROWS // ZROWS)
            ]
            for d in ds:
                d.wait()

        def _unpack(rb, sidx, aidx):
            for k in range(8):
                v = selr[pl.ds(rb + k * 16, 16)]
                sidx[pl.ds(k * 16, 16)] = v & SMASK
                aidx[pl.ds(k * 16, 16)] = lax.shift_right_logical(v, 14)

        def _wrap(p):
            return jnp.where(p >= CAP, p - CAP, p)

        _zero_stripe()
        plsc.subcore_barrier()

        def _chunk_body(cc, carry):
            chunk = cc * 2 + core
            lo = chunk * CN

            pend = pltpu.async_copy(
                ep_hbm.at[pl.ds(ebase, BLK)], st0, semt0)

            wp = jnp.int32(0)
            dp = jnp.int32(0)
            avail = jnp.int32(0)

            for b in range(NBLK):
                cur = st0 if b % 2 == 0 else st1
                pend.wait()
                if b + 1 < NBLK:
                    nxt = st1 if b % 2 == 0 else st0
                    pend = pltpu.async_copy(
                        ep_hbm.at[pl.ds(ebase + (b + 1) * BLK, BLK)],
                        nxt, semt1 if b % 2 == 0 else semt0)

                def _vec(i, st):
                    wp2, av2 = st
                    v = cur[pl.ds(i * 16, 16)]
                    sv = v & SMASK
                    rv = lax.shift_right_logical(v, 14) & 7
                    dv = lax.shift_right_logical(v, 17)
                    m = (dv >= lo) & (dv < lo + CN)
                    arow = rv * CN + (dv - lo)
                    packed = sv | (arow << 14)
                    mi = m.astype(jnp.int32)
                    pos = _wrap(wp2 + plsc.cumsum(mi) - 1)
                    plsc.store_scatter(selr, [pos], packed, mask=m)
                    cnt = jnp.sum(mi)
                    return (_wrap(wp2 + cnt), av2 + cnt)

                wp, avail = lax.fori_loop(0, BLK // 16, _vec, (wp, avail))

                def _quad(st):
                    dp2, av2 = st
                    gs = []
                    rb = dp2
                    for (sidx, aidx, row, semg, _) in slots:
                        _unpack(rb, sidx, aidx)
                        gs.append(
                            pltpu.async_copy(nf_hbm.at[sidx], row, semg))
                        rb = _wrap(rb + 128)
                    ss = []
                    for g, (sidx, aidx, row, _, sems) in zip(gs, slots):
                        g.wait()
                        ss.append(pltpu.async_copy(
                            row, acc.at[aidx], sems, add=True))
                    for s in ss:
                        s.wait()
                    return (rb, av2 - 512)

                dp, avail = lax.while_loop(
                    lambda st: st[1] >= 512, _quad, (dp, avail))

            def _single(st):
                dp2, av2 = st
                _unpack(dp2, sidx0, aidx0)
                pltpu.async_copy(nf_hbm.at[sidx0], row0, semg0).wait()
                pltpu.async_copy(row0, acc.at[aidx0], sems0, add=True).wait()
                return (_wrap(dp2 + 128), av2 - 128)

            dp, avail = lax.while_loop(
                lambda st: st[1] >= 128, _single, (dp, avail))

            for k in range(8):
                selr[pl.ds(wp + k * 16, 16)] = tpad

            @pl.when(avail > 0)
            def _():
                _unpack(dp, sidx0, aidx0)
                pltpu.async_copy(nf_hbm.at[sidx0], row0, semg0).wait()
                pltpu.async_copy(row0, acc.at[aidx0], sems0, add=True).wait()

            plsc.subcore_barrier()

            r_w = tid // 2
            j0 = (tid % 2) * WT_ROWS
            pltpu.sync_copy(acc.at[pl.ds(tid * WT_ROWS, WT_ROWS)],
                            a_hbm.at[chunk, r_w, pl.ds(j0, WT_ROWS)])

            _zero_stripe()
            plsc.subcore_barrier()
            return carry

        lax.fori_loop(0, NCHUNK // 2, _chunk_body, 0)

    return body


_sc_build = functools.partial(
    pl.kernel,
    out_type=jax.ShapeDtypeStruct((NCHUNK, R, CN, D), jnp.float32),
    mesh=plsc.VectorSubcoreMesh(core_axis_name="c", subcore_axis_name="s"),
    scratch_types=[
        pltpu.VMEM((BLK,), jnp.int32),
        pltpu.VMEM((BLK,), jnp.int32),
        pltpu.VMEM((CAP + 128,), jnp.int32),
    ] + [pltpu.VMEM((128,), jnp.int32) for _ in range(8)] + [
        pltpu.VMEM((128, D), jnp.float32),
        pltpu.VMEM((128, D), jnp.float32),
        pltpu.VMEM((128, D), jnp.float32),
        pltpu.VMEM((128, D), jnp.float32),
        pltpu.VMEM((ZROWS, D), jnp.float32),
        pltpu.VMEM_SHARED((ACC_ROWS, D), jnp.float32),
    ] + [pltpu.SemaphoreType.DMA for _ in range(11)],
    compiler_params=pltpu.CompilerParams(needs_layout_passes=False),
)(_make_sc_body())


ROWS_B = 320


def _tc_body(wc_ref, a_ref, x_ref, wb_ref, lw_ref, bias_ref,
             w1_ref, b1_ref, w2_ref, b2_ref, out_ref):
    x = x_ref[0]
    acc = jnp.dot(x, lw_ref[...], preferred_element_type=jnp.float32)
    for b in range(NB):
        bb = wc_ref[0, b] * a_ref[0, 0]
        for r in range(1, R):
            bb = bb + wc_ref[r, b] * a_ref[0, r]
        acc = acc + jnp.dot(bb, wb_ref[b], preferred_element_type=jnp.float32)
    acc = acc + bias_ref[...]
    h = jnp.maximum(
        jnp.dot(acc, w1_ref[...], preferred_element_type=jnp.float32)
        + b1_ref[...], 0.0)
    h = jnp.dot(h, w2_ref[...], preferred_element_type=jnp.float32) + b2_ref[...]
    out_ref[0] = jnp.maximum(h, 0.0)


def _tc_call(w_comp, a, xpad, weight, loop_w, bias, w1, b1, w2, b2):
    grid = (NCHUNK, CN // ROWS_B)
    full = lambda shape: pl.BlockSpec(shape, lambda c, j: (0,) * len(shape))
    return pl.pallas_call(
        _tc_body,
        grid=grid,
        in_specs=[
            pl.BlockSpec(memory_space=pltpu.SMEM),
            pl.BlockSpec((1, R, ROWS_B, D), lambda c, j: (c, 0, j, 0)),
            pl.BlockSpec((1, ROWS_B, D), lambda c, j: (c, j, 0)),
            full((NB, D, D)),
            full((D, D)),
            full((1, D)),
            full((D, D)),
            full((1, D)),
            full((D, D)),
            full((1, D)),
        ],
        out_specs=pl.BlockSpec((1, ROWS_B, D), lambda c, j: (c, j, 0)),
        out_shape=jax.ShapeDtypeStruct((NCHUNK, CN, D), jnp.float32),
    )(w_comp, a, xpad, weight, loop_w, bias, w1, b1, w2, b2)


def kernel(node_feat, edge_index, edge_type, weight, w_comp, loop_weight,
           bias, W1, b1, W2, b2):
    src = edge_index[0].astype(jnp.int32)
    dst = edge_index[1].astype(jnp.int32)
    rel = edge_type.astype(jnp.int32)
    epack = src | (rel << 14) | (dst << 17)

    a = _sc_build(node_feat, epack)

    xpad = jnp.pad(node_feat, ((0, NPAD - N), (0, 0))).reshape(NCHUNK, CN, D)
    out = _tc_call(w_comp, a, xpad, weight, loop_weight,
                   bias.reshape(1, D), W1, b1.reshape(1, D),
                   W2, b2.reshape(1, D))
    out = out.reshape(NPAD, D)[:N]
    return (out, edge_type)


# speedup vs baseline: 1.3842x; 1.3842x over previous
import functools

import jax
import jax.numpy as jnp
from jax import lax
from jax.experimental import pallas as pl
from jax.experimental.pallas import tpu as pltpu
from jax.experimental.pallas import tpu_sc as plsc

N = 10000
E = 320000
D = 128
R = 8
NB = 4

NPAD = 10240
NCHUNK = 8
CN = NPAD // NCHUNK
TRASH = R * CN
ACC_ROWS = R * CN + 8
EPT = E // 16
BLK = 2000
NBLK = EPT // BLK
BT = 160
CAP = 2400
WT_ROWS = R * CN // 16
SMASK = (1 << 14) - 1


def _make_sc_body():
    def body(nf_hbm, ep_hbm, a_hbm,
             st0, st1, selr,
             sidx0, aidxa0, aidxb0, sidx1, aidxa1, aidxb1,
             row0, row1, acc,
             semt0, semt1, semg0, semg1, sems0, sems1):
        core = lax.axis_index("c")
        tid = lax.axis_index("s")
        ebase = tid * EPT

        z16f = jnp.zeros((16,), jnp.float32)
        tpad = jnp.full((16,), TRASH << 14, jnp.int32)

        def _zrow(i, c):
            for k in range(8):
                row0[i, pl.ds(k * 16, 16)] = z16f
            return c

        def _zero_stripe():
            base = tid * WT_ROWS
            ds = [
                pltpu.async_copy(
                    row0, acc.at[pl.ds(base + k * BT, BT)], semg0)
                for k in range(WT_ROWS // BT)
            ]
            for d in ds:
                d.wait()

        def _unpack(rb, sidx, aidxa, aidxb):
            for k in range(BT // 16):
                v = selr[pl.ds(rb + k * 16, 16)]
                sidx[pl.ds(k * 16, 16)] = v & SMASK
                arow = lax.shift_right_logical(v, 14)
                if k < 8:
                    aidxa[pl.ds(k * 16, 16)] = arow
                else:
                    aidxb[pl.ds((k - 8) * 16, 16)] = arow

        def _scatter(row, aidxa, aidxb, sems):
            s1 = pltpu.async_copy(
                row.at[pl.ds(0, 128)], acc.at[aidxa], sems, add=True)
            s2 = pltpu.async_copy(
                row.at[pl.ds(128, BT - 128)], acc.at[aidxb], sems, add=True)
            return (s1, s2)

        def _wrap(p):
            return jnp.where(p >= CAP, p - CAP, p)

        lax.fori_loop(0, BT, _zrow, 0)
        _zero_stripe()
        plsc.subcore_barrier()

        def _chunk_body(cc, carry):
            chunk = cc * 2 + core
            lo = chunk * CN

            pend = pltpu.async_copy(
                ep_hbm.at[pl.ds(ebase, BLK)], st0, semt0)

            wp = jnp.int32(0)
            dp = jnp.int32(0)
            avail = jnp.int32(0)

            for b in range(NBLK):
                cur = st0 if b % 2 == 0 else st1
                pend.wait()
                if b + 1 < NBLK:
                    nxt = st1 if b % 2 == 0 else st0
                    pend = pltpu.async_copy(
                        ep_hbm.at[pl.ds(ebase + (b + 1) * BLK, BLK)],
                        nxt, semt1 if b % 2 == 0 else semt0)

                def _vec(i, st):
                    wp2, av2 = st
                    v = cur[pl.ds(i * 16, 16)]
                    sv = v & SMASK
                    rv = lax.shift_right_logical(v, 14) & 7
                    dv = lax.shift_right_logical(v, 17)
                    m = (dv >= lo) & (dv < lo + CN)
                    arow = rv * CN + (dv - lo)
                    packed = sv | (arow << 14)
                    mi = m.astype(jnp.int32)
                    pos = _wrap(wp2 + plsc.cumsum(mi) - 1)
                    plsc.store_scatter(selr, [pos], packed, mask=m)
                    cnt = jnp.sum(mi)
                    return (_wrap(wp2 + cnt), av2 + cnt)

                wp, avail = lax.fori_loop(0, BLK // 16, _vec, (wp, avail))

                def _pair(st):
                    dp2, av2 = st
                    rb0 = dp2
                    rb1 = _wrap(dp2 + BT)
                    _unpack(rb0, sidx0, aidxa0, aidxb0)
                    g0 = pltpu.async_copy(nf_hbm.at[sidx0], row0, semg0)
                    _unpack(rb1, sidx1, aidxa1, aidxb1)
                    g1 = pltpu.async_copy(nf_hbm.at[sidx1], row1, semg1)
                    g0.wait()
                    s0 = _scatter(row0, aidxa0, aidxb0, sems0)
                    g1.wait()
                    s1 = _scatter(row1, aidxa1, aidxb1, sems1)
                    for s in s0 + s1:
                        s.wait()
                    return (_wrap(_wrap(dp2 + BT) + BT), av2 - 2 * BT)

                dp, avail = lax.while_loop(
                    lambda st: st[1] >= 2 * BT, _pair, (dp, avail))

            def _single(st):
                dp2, av2 = st
                _unpack(dp2, sidx0, aidxa0, aidxb0)
                pltpu.async_copy(nf_hbm.at[sidx0], row0, semg0).wait()
                for s in _scatter(row0, aidxa0, aidxb0, sems0):
                    s.wait()
                return (_wrap(dp2 + BT), av2 - BT)

            dp, avail = lax.while_loop(
                lambda st: st[1] >= BT, _single, (dp, avail))

            for k in range(BT // 16):
                selr[pl.ds(wp + k * 16, 16)] = tpad

            @pl.when(avail > 0)
            def _():
                _unpack(dp, sidx0, aidxa0, aidxb0)
                pltpu.async_copy(nf_hbm.at[sidx0], row0, semg0).wait()
                for s in _scatter(row0, aidxa0, aidxb0, sems0):
                    s.wait()

            plsc.subcore_barrier()

            r_w = tid // 2
            j0 = (tid % 2) * WT_ROWS
            pltpu.sync_copy(acc.at[pl.ds(tid * WT_ROWS, WT_ROWS)],
                            a_hbm.at[chunk, r_w, pl.ds(j0, WT_ROWS)])

            lax.fori_loop(0, BT, _zrow, 0)
            _zero_stripe()
            plsc.subcore_barrier()
            return carry

        lax.fori_loop(0, NCHUNK // 2, _chunk_body, 0)

    return body


_sc_build = functools.partial(
    pl.kernel,
    out_type=jax.ShapeDtypeStruct((NCHUNK, R, CN, D), jnp.float32),
    mesh=plsc.VectorSubcoreMesh(core_axis_name="c", subcore_axis_name="s"),
    scratch_types=[
        pltpu.VMEM((BLK,), jnp.int32),
        pltpu.VMEM((BLK,), jnp.int32),
        pltpu.VMEM((CAP + BT,), jnp.int32),
        pltpu.VMEM((BT,), jnp.int32),
        pltpu.VMEM((128,), jnp.int32),
        pltpu.VMEM((BT - 128,), jnp.int32),
        pltpu.VMEM((BT,), jnp.int32),
        pltpu.VMEM((128,), jnp.int32),
        pltpu.VMEM((BT - 128,), jnp.int32),
        pltpu.VMEM((BT, D), jnp.float32),
        pltpu.VMEM((BT, D), jnp.float32),
        pltpu.VMEM_SHARED((ACC_ROWS, D), jnp.float32),
    ] + [pltpu.SemaphoreType.DMA for _ in range(6)],
    compiler_params=pltpu.CompilerParams(needs_layout_passes=False),
)(_make_sc_body())


ROWS_B = 256


def _tc_body(wc_ref, a_ref, x_ref, wb_ref, lw_ref, bias_ref,
             w1_ref, b1_ref, w2_ref, b2_ref, out_ref):
    x = x_ref[0]
    acc = jnp.dot(x, lw_ref[...], preferred_element_type=jnp.float32)
    for b in range(NB):
        bb = wc_ref[0, b] * a_ref[0, 0]
        for r in range(1, R):
            bb = bb + wc_ref[r, b] * a_ref[0, r]
        acc = acc + jnp.dot(bb, wb_ref[b], preferred_element_type=jnp.float32)
    acc = acc + bias_ref[...]
    h = jnp.maximum(
        jnp.dot(acc, w1_ref[...], preferred_element_type=jnp.float32)
        + b1_ref[...], 0.0)
    h = jnp.dot(h, w2_ref[...], preferred_element_type=jnp.float32) + b2_ref[...]
    out_ref[0] = jnp.maximum(h, 0.0)


def _tc_call(w_comp, a, xpad, weight, loop_w, bias, w1, b1, w2, b2):
    grid = (NCHUNK, CN // ROWS_B)
    full = lambda shape: pl.BlockSpec(shape, lambda c, j: (0,) * len(shape))
    return pl.pallas_call(
        _tc_body,
        grid=grid,
        in_specs=[
            pl.BlockSpec(memory_space=pltpu.SMEM),
            pl.BlockSpec((1, R, ROWS_B, D), lambda c, j: (c, 0, j, 0)),
            pl.BlockSpec((1, ROWS_B, D), lambda c, j: (c, j, 0)),
            full((NB, D, D)),
            full((D, D)),
            full((1, D)),
            full((D, D)),
            full((1, D)),
            full((D, D)),
            full((1, D)),
        ],
        out_specs=pl.BlockSpec((1, ROWS_B, D), lambda c, j: (c, j, 0)),
        out_shape=jax.ShapeDtypeStruct((NCHUNK, CN, D), jnp.float32),
    )(w_comp, a, xpad, weight, loop_w, bias, w1, b1, w2, b2)


def kernel(node_feat, edge_index, edge_type, weight, w_comp, loop_weight,
           bias, W1, b1, W2, b2):
    src = edge_index[0].astype(jnp.int32)
    dst = edge_index[1].astype(jnp.int32)
    rel = edge_type.astype(jnp.int32)
    epack = src | (rel << 14) | (dst << 17)

    a = _sc_build(node_feat, epack)

    xpad = jnp.pad(node_feat, ((0, NPAD - N), (0, 0))).reshape(NCHUNK, CN, D)
    out = _tc_call(w_comp, a, xpad, weight, loop_weight,
                   bias.reshape(1, D), W1, b1.reshape(1, D),
                   W2, b2.reshape(1, D))
    out = out.reshape(NPAD, D)[:N]
    return (out, edge_type)


# speedup vs baseline: 1.5739x; 1.1370x over previous
import functools

import jax
import jax.numpy as jnp
from jax import lax
from jax.experimental import pallas as pl
from jax.experimental.pallas import tpu as pltpu
from jax.experimental.pallas import tpu_sc as plsc

N = 10000
E = 320000
D = 128
R = 8
NB = 4

NPAD = 10240
NCHUNK = 8
CN = NPAD // NCHUNK
TRASH = R * CN
ACC_ROWS = R * CN + 128
EPT = E // 16
BLK = 2000
NBLK = EPT // BLK
CAP = 2304
WT_ROWS = R * CN // 16
ZROWS = 40
SMASK = (1 << 14) - 1


def _make_sc_body(phase):
    def body(nf_hbm, ep_hbm, a_hbm,
             st0, st1, selr, sidx0, aidx0, sidx1, aidx1,
             row0, row1, zbuf, acc,
             semt0, semt1, semg0, semg1, sems0, sems1, semz):
        core = lax.axis_index("c")
        tid = lax.axis_index("s")
        ebase = tid * EPT

        z16f = jnp.zeros((16,), jnp.float32)
        tpad = jnp.full((16,), TRASH << 14, jnp.int32)

        def _zrow(i, c):
            for k in range(8):
                zbuf[i, pl.ds(k * 16, 16)] = z16f
            return c
        lax.fori_loop(0, ZROWS, _zrow, 0)

        def _zero_stripe():
            base = tid * WT_ROWS
            ds = [
                pltpu.async_copy(
                    zbuf, acc.at[pl.ds(base + k * ZROWS, ZROWS)], semz)
                for k in range(WT_ROWS // ZROWS)
            ]
            for d in ds:
                d.wait()

        def _unpack(rb, sidx, aidx):
            for k in range(8):
                v = selr[pl.ds(rb + k * 16, 16)]
                sidx[pl.ds(k * 16, 16)] = v & SMASK
                aidx[pl.ds(k * 16, 16)] = lax.shift_right_logical(v, 14)

        def _wrap(p):
            return jnp.where(p >= CAP, p - CAP, p)

        _zero_stripe()
        plsc.subcore_barrier()

        def _chunk_body(cc, carry):
            lchunk = cc * 2 + core
            chunk = phase * (NCHUNK // 2) + lchunk
            lo = chunk * CN

            pend = pltpu.async_copy(
                ep_hbm.at[pl.ds(ebase, BLK)], st0, semt0)

            wp = jnp.int32(0)
            dp = jnp.int32(0)
            avail = jnp.int32(0)

            for b in range(NBLK):
                cur = st0 if b % 2 == 0 else st1
                pend.wait()
                if b + 1 < NBLK:
                    nxt = st1 if b % 2 == 0 else st0
                    pend = pltpu.async_copy(
                        ep_hbm.at[pl.ds(ebase + (b + 1) * BLK, BLK)],
                        nxt, semt1 if b % 2 == 0 else semt0)

                def _vec(i, st):
                    wp2, av2 = st
                    v = cur[pl.ds(i * 16, 16)]
                    sv = v & SMASK
                    rv = lax.shift_right_logical(v, 14) & 7
                    dv = lax.shift_right_logical(v, 17)
                    m = (dv >= lo) & (dv < lo + CN)
                    arow = rv * CN + (dv - lo)
                    packed = sv | (arow << 14)
                    mi = m.astype(jnp.int32)
                    pos = _wrap(wp2 + plsc.cumsum(mi) - 1)
                    plsc.store_scatter(selr, [pos], packed, mask=m)
                    cnt = jnp.sum(mi)
                    return (_wrap(wp2 + cnt), av2 + cnt)

                wp, avail = lax.fori_loop(0, BLK // 16, _vec, (wp, avail))

                def _pair(st):
                    dp2, av2 = st
                    rb0 = dp2
                    rb1 = _wrap(dp2 + 128)
                    _unpack(rb0, sidx0, aidx0)
                    g0 = pltpu.async_copy(nf_hbm.at[sidx0], row0, semg0)
                    _unpack(rb1, sidx1, aidx1)
                    g1 = pltpu.async_copy(nf_hbm.at[sidx1], row1, semg1)
                    g0.wait()
                    s0 = pltpu.async_copy(row0, acc.at[aidx0], sems0, add=True)
                    g1.wait()
                    s1 = pltpu.async_copy(row1, acc.at[aidx1], sems1, add=True)
                    s0.wait()
                    s1.wait()
                    return (_wrap(_wrap(dp2 + 128) + 128), av2 - 256)

                dp, avail = lax.while_loop(
                    lambda st: st[1] >= 256, _pair, (dp, avail))

            @pl.when(avail >= 128)
            def _():
                _unpack(dp, sidx0, aidx0)
                pltpu.async_copy(nf_hbm.at[sidx0], row0, semg0).wait()
                pltpu.async_copy(row0, acc.at[aidx0], sems0, add=True).wait()

            dp = jnp.where(avail >= 128, _wrap(dp + 128), dp)
            avail = avail - jnp.where(avail >= 128, 128, 0)

            for k in range(8):
                selr[pl.ds(wp + k * 16, 16)] = tpad

            @pl.when(avail > 0)
            def _():
                _unpack(dp, sidx0, aidx0)
                pltpu.async_copy(nf_hbm.at[sidx0], row0, semg0).wait()
                pltpu.async_copy(row0, acc.at[aidx0], sems0, add=True).wait()

            plsc.subcore_barrier()

            r_w = tid // 2
            j0 = (tid % 2) * WT_ROWS
            pltpu.sync_copy(acc.at[pl.ds(tid * WT_ROWS, WT_ROWS)],
                            a_hbm.at[lchunk, r_w, pl.ds(j0, WT_ROWS)])

            _zero_stripe()
            plsc.subcore_barrier()
            return carry

        lax.fori_loop(0, NCHUNK // 4, _chunk_body, 0)

    return body


def _make_sc_build(phase):
    return functools.partial(
        pl.kernel,
        out_type=jax.ShapeDtypeStruct((NCHUNK // 2, R, CN, D), jnp.float32),
    mesh=plsc.VectorSubcoreMesh(core_axis_name="c", subcore_axis_name="s"),
    scratch_types=[
        pltpu.VMEM((BLK,), jnp.int32),
        pltpu.VMEM((BLK,), jnp.int32),
        pltpu.VMEM((CAP + 128,), jnp.int32),
        pltpu.VMEM((128,), jnp.int32),
        pltpu.VMEM((128,), jnp.int32),
        pltpu.VMEM((128,), jnp.int32),
        pltpu.VMEM((128,), jnp.int32),
        pltpu.VMEM((128, D), jnp.float32),
        pltpu.VMEM((128, D), jnp.float32),
        pltpu.VMEM((ZROWS, D), jnp.float32),
        pltpu.VMEM_SHARED((ACC_ROWS, D), jnp.float32),
        pltpu.SemaphoreType.DMA,
        pltpu.SemaphoreType.DMA,
        pltpu.SemaphoreType.DMA,
        pltpu.SemaphoreType.DMA,
        pltpu.SemaphoreType.DMA,
        pltpu.SemaphoreType.DMA,
        pltpu.SemaphoreType.DMA,
    ],
        compiler_params=pltpu.CompilerParams(needs_layout_passes=False),
        name=f"sc_rgin_phase{phase}",
    )(_make_sc_body(phase))


_sc_builds = [_make_sc_build(0), _make_sc_build(1)]


ROWS_B = 256


def _tc_body(wc_ref, a_ref, x_ref, wb_ref, lw_ref, bias_ref,
             w1_ref, b1_ref, w2_ref, b2_ref, out_ref):
    x = x_ref[0]
    acc = jnp.dot(x, lw_ref[...], preferred_element_type=jnp.float32)
    for b in range(NB):
        bb = wc_ref[0, b] * a_ref[0, 0]
        for r in range(1, R):
            bb = bb + wc_ref[r, b] * a_ref[0, r]
        acc = acc + jnp.dot(bb, wb_ref[b], preferred_element_type=jnp.float32)
    acc = acc + bias_ref[...]
    h = jnp.maximum(
        jnp.dot(acc, w1_ref[...], preferred_element_type=jnp.float32)
        + b1_ref[...], 0.0)
    h = jnp.dot(h, w2_ref[...], preferred_element_type=jnp.float32) + b2_ref[...]
    out_ref[0] = jnp.maximum(h, 0.0)


def _tc_call(w_comp, a, xpad, weight, loop_w, bias, w1, b1, w2, b2):
    grid = (NCHUNK // 2, CN // ROWS_B)
    full = lambda shape: pl.BlockSpec(shape, lambda c, j: (0,) * len(shape))
    return pl.pallas_call(
        _tc_body,
        grid=grid,
        in_specs=[
            pl.BlockSpec(memory_space=pltpu.SMEM),
            pl.BlockSpec((1, R, ROWS_B, D), lambda c, j: (c, 0, j, 0)),
            pl.BlockSpec((1, ROWS_B, D), lambda c, j: (c, j, 0)),
            full((NB, D, D)),
            full((D, D)),
            full((1, D)),
            full((D, D)),
            full((1, D)),
            full((D, D)),
            full((1, D)),
        ],
        out_specs=pl.BlockSpec((1, ROWS_B, D), lambda c, j: (c, j, 0)),
        out_shape=jax.ShapeDtypeStruct((NCHUNK // 2, CN, D), jnp.float32),
    )(w_comp, a, xpad, weight, loop_w, bias, w1, b1, w2, b2)


def kernel(node_feat, edge_index, edge_type, weight, w_comp, loop_weight,
           bias, W1, b1, W2, b2):
    src = edge_index[0].astype(jnp.int32)
    dst = edge_index[1].astype(jnp.int32)
    rel = edge_type.astype(jnp.int32)
    epack = src | (rel << 14) | (dst << 17)

    a0 = _sc_builds[0](node_feat, epack)
    a1 = _sc_builds[1](node_feat, epack)

    xpad = jnp.pad(node_feat, ((0, NPAD - N), (0, 0))).reshape(NCHUNK, CN, D)
    wargs = (weight, loop_weight, bias.reshape(1, D), W1, b1.reshape(1, D),
             W2, b2.reshape(1, D))
    out0 = _tc_call(w_comp, a0, xpad[:NCHUNK // 2], *wargs)
    out1 = _tc_call(w_comp, a1, xpad[NCHUNK // 2:], *wargs)
    out = jnp.concatenate([out0, out1]).reshape(NPAD, D)[:N]
    return (out, edge_type)
